# baseline (speedup 1.0000x reference)
import jax
import jax.numpy as jnp
from jax import lax
from jax.experimental import pallas as pl
from jax.experimental.pallas import tpu as pltpu
from jax.experimental.pallas import tpu_sc as plsc

NC = 2
NS = 16
L = 16
NW = NC * NS
B = 16384
D = 32
N = 1000000
PLANE = 1000064
BPW = B // NW
NG = BPW // L
CH = 128
NCH = BPW // CH


def _sc_body(iso_hbm, rand_hbm, idx1_hbm, idx2_hbm, beta_hbm, out_hbm,
             idx1_v, idx2_v, za1, za2, zb1, zb2, r1_v, r2_v,
             acc_v, beta_v, out_v, sem_a, sem_b, sem_r):
    wid = lax.axis_index("s") * NC + lax.axis_index("c")

    pltpu.sync_copy(idx1_hbm.at[wid], idx1_v)
    pltpu.sync_copy(idx2_hbm.at[wid], idx2_v)
    pltpu.sync_copy(beta_hbm, beta_v)

    rcopies = []
    for j in range(NCH):
        sl = pl.ds(j * CH, CH)
        rcopies.append(pltpu.async_copy(
            rand_hbm.at[idx1_v.at[pl.ds(j * CH, CH)]], r1_v.at[sl], sem_r))
        rcopies.append(pltpu.async_copy(
            rand_hbm.at[idx2_v.at[pl.ds(j * CH, CH)]], r2_v.at[sl], sem_r))

    def fire(d, bufs, sem):
        cs = []
        src = iso_hbm.at[d]
        z1b, z2b = bufs
        for j in range(NCH):
            sl = pl.ds(j * CH, CH)
            cs.append(pltpu.async_copy(
                src.at[idx1_v.at[pl.ds(j * CH, CH)]], z1b.at[sl], sem))
            cs.append(pltpu.async_copy(
                src.at[idx2_v.at[pl.ds(j * CH, CH)]], z2b.at[sl], sem))
        return cs

    def zero_group(g, carry):
        acc_v[pl.ds(g * L, L)] = jnp.zeros((L,), jnp.float32)
        return carry

    lax.fori_loop(0, NG, zero_group, 0)

    bufs = ((za1, za2), (zb1, zb2))
    sems = (sem_a, sem_b)
    pend = fire(0, bufs[0], sems[0])
    for d in range(D):
        if d + 1 < D:
            nxt = fire(d + 1, bufs[(d + 1) % 2], sems[(d + 1) % 2])
        for c in pend:
            c.wait()
        pend = nxt if d + 1 < D else []
        z1b, z2b = bufs[d % 2]

        def accum(g, carry):
            sl = pl.ds(g * L, L)
            df = z1b[sl] - z2b[sl]
            acc_v[sl] = acc_v[sl] + df * df
            return carry

        lax.fori_loop(0, NG, accum, 0)

    for c in rcopies:
        c.wait()

    beta_vec = beta_v[...]

    def finish(g, carry):
        sl = pl.ds(g * L, L)
        acc = acc_v[sl]
        seed = jnp.int32(0x5F3759DF) - (plsc.bitcast(acc, jnp.int32) >> 1)
        y = plsc.bitcast(seed, jnp.float32)
        h = acc * jnp.float32(0.5)
        for _ in range(3):
            y = y * (jnp.float32(1.5) - h * y * y)
        dist = acc * y
        out_v[sl] = r1_v[sl] + r2_v[sl] - beta_vec * dist
        return carry

    lax.fori_loop(0, NG, finish, 0)
    pltpu.sync_copy(out_v, out_hbm.at[pl.ds(wid * BPW, BPW)])


def kernel(protein1_idx, protein2_idx, iso_emb, rand_eff, beta_iso):
    idx1 = protein1_idx.astype(jnp.int32).reshape(NW, BPW)
    idx2 = protein2_idx.astype(jnp.int32).reshape(NW, BPW)
    isof = iso_emb.astype(jnp.float32)
    iso_t = jnp.stack([isof[:, d] for d in range(D)], axis=0)
    rand_flat = rand_eff.astype(jnp.float32).reshape(N)
    beta = jnp.full((L,), beta_iso, jnp.float32)
    mesh = plsc.VectorSubcoreMesh(
        core_axis_name="c", subcore_axis_name="s",
        num_cores=NC, num_subcores=NS)
    run = pl.kernel(
        _sc_body,
        out_type=jax.ShapeDtypeStruct((B,), jnp.float32),
        mesh=mesh,
        compiler_params=pltpu.CompilerParams(
            needs_layout_passes=False, use_tc_tiling_on_sc=False),
        scratch_types=[
            pltpu.VMEM((BPW,), jnp.int32),
            pltpu.VMEM((BPW,), jnp.int32),
            pltpu.VMEM((BPW,), jnp.float32),
            pltpu.VMEM((BPW,), jnp.float32),
            pltpu.VMEM((BPW,), jnp.float32),
            pltpu.VMEM((BPW,), jnp.float32),
            pltpu.VMEM((BPW,), jnp.float32),
            pltpu.VMEM((BPW,), jnp.float32),
            pltpu.VMEM((BPW,), jnp.float32),
            pltpu.VMEM((L,), jnp.float32),
            pltpu.VMEM((BPW,), jnp.float32),
            pltpu.SemaphoreType.DMA,
            pltpu.SemaphoreType.DMA,
            pltpu.SemaphoreType.DMA,
        ],
    )
    return run(iso_t, rand_flat, idx1, idx2, beta)


# speedup vs baseline: 1.2873x; 1.2873x over previous
import jax
import jax.numpy as jnp
from jax import lax
from jax.experimental import pallas as pl
from jax.experimental.pallas import tpu as pltpu
from jax.experimental.pallas import tpu_sc as plsc

NC = 2
NS = 16
L = 16
NW = NC * NS
B = 16384
D = 32
N = 1000000
PLANE = 1000064
BPW = B // NW
NG = BPW // L
CH = 128
NCH = BPW // CH


def _sc_body(iso_hbm, rand_hbm, idx1_hbm, idx2_hbm, beta_hbm, out_hbm,
             idx1_v, idx2_v, za1, za2, zb1, zb2, r1_v, r2_v,
             acc_v, beta_v, out_v, sem_a, sem_b, sem_r):
    wid = lax.axis_index("s") * NC + lax.axis_index("c")

    pltpu.sync_copy(idx1_hbm.at[wid], idx1_v)
    pltpu.sync_copy(idx2_hbm.at[wid], idx2_v)
    pltpu.sync_copy(beta_hbm, beta_v)

    rcopies = []
    for j in range(NCH):
        sl = pl.ds(j * CH, CH)
        rcopies.append(pltpu.async_copy(
            rand_hbm.at[idx1_v.at[pl.ds(j * CH, CH)]], r1_v.at[sl], sem_r))
        rcopies.append(pltpu.async_copy(
            rand_hbm.at[idx2_v.at[pl.ds(j * CH, CH)]], r2_v.at[sl], sem_r))

    def fire(d, bufs, sem):
        cs = []
        src = iso_hbm.at[d]
        z1b, z2b = bufs
        for j in range(NCH):
            sl = pl.ds(j * CH, CH)
            cs.append(pltpu.async_copy(
                src.at[idx1_v.at[pl.ds(j * CH, CH)]], z1b.at[sl], sem))
            cs.append(pltpu.async_copy(
                src.at[idx2_v.at[pl.ds(j * CH, CH)]], z2b.at[sl], sem))
        return cs

    def zero_group(g, carry):
        acc_v[pl.ds(g * L, L)] = jnp.zeros((L,), jnp.float32)
        return carry

    lax.fori_loop(0, NG, zero_group, 0)

    bufs = ((za1, za2), (zb1, zb2))
    sems = (sem_a, sem_b)
    pend = fire(0, bufs[0], sems[0])
    for d in range(D):
        if d + 1 < D:
            nxt = fire(d + 1, bufs[(d + 1) % 2], sems[(d + 1) % 2])
        for c in pend:
            c.wait()
        pend = nxt if d + 1 < D else []
        z1b, z2b = bufs[d % 2]

        def accum(g, carry):
            sl = pl.ds(g * L, L)
            df = z1b[sl] - z2b[sl]
            acc_v[sl] = acc_v[sl] + df * df
            return carry

        lax.fori_loop(0, NG, accum, 0)

    for c in rcopies:
        c.wait()

    beta_vec = beta_v[...]

    def finish(g, carry):
        sl = pl.ds(g * L, L)
        acc = acc_v[sl]
        seed = jnp.int32(0x5F3759DF) - (plsc.bitcast(acc, jnp.int32) >> 1)
        y = plsc.bitcast(seed, jnp.float32)
        h = acc * jnp.float32(0.5)
        for _ in range(3):
            y = y * (jnp.float32(1.5) - h * y * y)
        dist = acc * y
        out_v[sl] = r1_v[sl] + r2_v[sl] - beta_vec * dist
        return carry

    lax.fori_loop(0, NG, finish, 0)
    pltpu.sync_copy(out_v, out_hbm.at[pl.ds(wid * BPW, BPW)])


def kernel(protein1_idx, protein2_idx, iso_emb, rand_eff, beta_iso):
    idx1 = protein1_idx.astype(jnp.int32).reshape(NW, BPW)
    idx2 = protein2_idx.astype(jnp.int32).reshape(NW, BPW)
    iso_t = iso_emb.astype(jnp.float32).T
    rand_flat = rand_eff.astype(jnp.float32).reshape(N)
    beta = jnp.full((L,), beta_iso, jnp.float32)
    mesh = plsc.VectorSubcoreMesh(
        core_axis_name="c", subcore_axis_name="s",
        num_cores=NC, num_subcores=NS)
    run = pl.kernel(
        _sc_body,
        out_type=jax.ShapeDtypeStruct((B,), jnp.float32),
        mesh=mesh,
        compiler_params=pltpu.CompilerParams(
            needs_layout_passes=False, use_tc_tiling_on_sc=False),
        scratch_types=[
            pltpu.VMEM((BPW,), jnp.int32),
            pltpu.VMEM((BPW,), jnp.int32),
            pltpu.VMEM((BPW,), jnp.float32),
            pltpu.VMEM((BPW,), jnp.float32),
            pltpu.VMEM((BPW,), jnp.float32),
            pltpu.VMEM((BPW,), jnp.float32),
            pltpu.VMEM((BPW,), jnp.float32),
            pltpu.VMEM((BPW,), jnp.float32),
            pltpu.VMEM((BPW,), jnp.float32),
            pltpu.VMEM((L,), jnp.float32),
            pltpu.VMEM((BPW,), jnp.float32),
            pltpu.SemaphoreType.DMA,
            pltpu.SemaphoreType.DMA,
            pltpu.SemaphoreType.DMA,
        ],
    )
    return run(iso_t, rand_flat, idx1, idx2, beta)


# speedup vs baseline: 6.4072x; 4.9774x over previous
import jax
import jax.numpy as jnp
from jax import lax
from jax.experimental import pallas as pl
from jax.experimental.pallas import tpu as pltpu
from jax.experimental.pallas import tpu_sc as plsc

NC = 2
NS = 16
L = 16
NW = NC * NS
B = 16384
D = 32
N = 1000000
BPW = B // NW
NG = BPW // L
CH = 128
NCH = BPW // CH


def _sc_body(iso_hbm, rand_hbm, idx1_hbm, idx2_hbm, beta_hbm, out_hbm,
             idx1_v, idx2_v, z1_v, z2_v, r1_v, r2_v, beta_v, out_v,
             sem_z, sem_r):
    wid = lax.axis_index("s") * NC + lax.axis_index("c")

    pltpu.sync_copy(idx1_hbm.at[wid], idx1_v)
    pltpu.sync_copy(idx2_hbm.at[wid], idx2_v)
    pltpu.sync_copy(beta_hbm, beta_v)

    copies = []
    for j in range(NCH):
        sl = pl.ds(j * CH, CH)
        copies.append(pltpu.async_copy(
            iso_hbm.at[idx1_v.at[pl.ds(j * CH, CH)]], z1_v.at[sl], sem_z))
        copies.append(pltpu.async_copy(
            iso_hbm.at[idx2_v.at[pl.ds(j * CH, CH)]], z2_v.at[sl], sem_z))
        copies.append(pltpu.async_copy(
            rand_hbm.at[idx1_v.at[pl.ds(j * CH, CH)]], r1_v.at[sl], sem_r))
        copies.append(pltpu.async_copy(
            rand_hbm.at[idx2_v.at[pl.ds(j * CH, CH)]], r2_v.at[sl], sem_r))
    for c in copies:
        c.wait()

    beta_vec = beta_v[...]
    iota = lax.iota(jnp.int32, L)

    def group(g, carry):
        sl = pl.ds(g * L, L)
        rows = g * L + iota
        acc = jnp.zeros((L,), jnp.float32)
        for d in range(D):
            col = jnp.full((L,), d, jnp.int32)
            a = plsc.load_gather(z1_v, [rows, col])
            b = plsc.load_gather(z2_v, [rows, col])
            df = a - b
            acc = acc + df * df
        seed = jnp.int32(0x5F3759DF) - (plsc.bitcast(acc, jnp.int32) >> 1)
        y = plsc.bitcast(seed, jnp.float32)
        h = acc * jnp.float32(0.5)
        for _ in range(3):
            y = y * (jnp.float32(1.5) - h * y * y)
        dist = acc * y
        out_v[sl] = r1_v[sl] + r2_v[sl] - beta_vec * dist
        return carry

    lax.fori_loop(0, NG, group, 0)
    pltpu.sync_copy(out_v, out_hbm.at[pl.ds(wid * BPW, BPW)])


def kernel(protein1_idx, protein2_idx, iso_emb, rand_eff, beta_iso):
    idx1 = protein1_idx.astype(jnp.int32).reshape(NW, BPW)
    idx2 = protein2_idx.astype(jnp.int32).reshape(NW, BPW)
    rand_flat = rand_eff.astype(jnp.float32).reshape(N)
    beta = jnp.full((L,), beta_iso, jnp.float32)
    mesh = plsc.VectorSubcoreMesh(
        core_axis_name="c", subcore_axis_name="s",
        num_cores=NC, num_subcores=NS)
    run = pl.kernel(
        _sc_body,
        out_type=jax.ShapeDtypeStruct((B,), jnp.float32),
        mesh=mesh,
        compiler_params=pltpu.CompilerParams(
            needs_layout_passes=False, use_tc_tiling_on_sc=False),
        scratch_types=[
            pltpu.VMEM((BPW,), jnp.int32),
            pltpu.VMEM((BPW,), jnp.int32),
            pltpu.VMEM((BPW, D), jnp.float32),
            pltpu.VMEM((BPW, D), jnp.float32),
            pltpu.VMEM((BPW,), jnp.float32),
            pltpu.VMEM((BPW,), jnp.float32),
            pltpu.VMEM((L,), jnp.float32),
            pltpu.VMEM((BPW,), jnp.float32),
            pltpu.SemaphoreType.DMA,
            pltpu.SemaphoreType.DMA,
        ],
    )
    return run(iso_emb.astype(jnp.float32), rand_flat, idx1, idx2, beta)
